# baseline (speedup 1.0000x reference)
import functools

import jax
import jax.numpy as jnp
from jax import lax
from jax.experimental import pallas as pl
from jax.experimental.pallas import tpu as pltpu
from jax.experimental.pallas import tpu_sc as plsc

KF = 512
KB = 128
HPC = KB * 16 + KF
LANES = 16


def _sc_histogram_kernel(C, HW, PPW, S, pf_hbm, lf_hbm, out_hbm,
                         prob_a, prob_b, lab_a, lab_b, sem_a, sem_b, *hists):
    cid = lax.axis_index("c")
    sid = lax.axis_index("s")
    wid = sid * 2 + cid
    base = wid * PPW
    wpb = HW // PPW
    b = wid // wpb
    off = base - b * HW

    row0 = b * C

    for hc in hists:
        def zero_body(i, _, hc=hc):
            hc[pl.ds(i * LANES, LANES)] = jnp.zeros((LANES,), jnp.float32)
            return 0

        lax.fori_loop(0, HPC // LANES, zero_body, 0)

    SQ = S // 128
    q0 = off // 128

    def issue(t, prob_v, lab_v, sem):
        pltpu.async_copy(lf_hbm.at[pl.ds(base + t * S, S)], lab_v, sem)
        qt = pl.multiple_of(q0 + t * SQ, 8)
        for c in range(C):
            pltpu.async_copy(pf_hbm.at[row0 + c, pl.ds(qt, SQ), :],
                             prob_v.at[pl.ds(c * SQ, SQ), :], sem)

    def drain(prob_v, lab_v, sem):
        pltpu.make_async_copy(lf_hbm.at[pl.ds(0, S)], lab_v, sem).wait()
        pltpu.make_async_copy(
            pf_hbm.at[0, pl.ds(0, C * SQ), :], prob_v, sem).wait()

    ones = jnp.ones((LANES,), jnp.float32)
    lane = lax.broadcasted_iota(jnp.int32, (LANES,), 0)
    nvec = S // LANES

    def compute(prob_v, lab_v):
        def vec_body(v, _):
            lab = lab_v[pl.ds(v * LANES, LANES)]
            qrow = v >> 3
            qcol = (v & 7) * LANES
            idxs = []
            for c in range(C):
                p = prob_v[c * SQ + qrow, pl.ds(qcol, LANES)]
                fg = lab == c
                bb = (p * float(KF)).astype(jnp.int32)
                idx_bg = ((bb >> 2) << 4) + lane
                idx_fg = (KB * 16 + KF - 1) - bb
                idxs.append(jnp.where(fg, idx_fg, idx_bg))
            for c in range(C):
                plsc.addupdate_scatter(hists[c], [idxs[c]], ones)
            return 0

        lax.fori_loop(0, nvec, vec_body, 0)

    NCH = PPW // S
    issue(0, prob_a, lab_a, sem_a)

    def pair_body(i, _):
        t0 = 2 * i
        issue(t0 + 1, prob_b, lab_b, sem_b)
        drain(prob_a, lab_a, sem_a)
        compute(prob_a, lab_a)

        @pl.when(t0 + 2 < NCH)
        def _prefetch():
            issue(t0 + 2, prob_a, lab_a, sem_a)

        drain(prob_b, lab_b, sem_b)
        compute(prob_b, lab_b)
        return 0

    lax.fori_loop(0, NCH // 2, pair_body, 0)
    for c, hc in enumerate(hists):
        pltpu.sync_copy(hc, out_hbm.at[pl.ds(wid * C * HPC + c * HPC, HPC)])


def _finalize_kernel(C, bg_ref, fg_ref, out_ref):
    bgs = jnp.sum(bg_ref[...], axis=0)
    bg = jnp.sum(bgs, axis=-1)
    fg = jnp.sum(fg_ref[...], axis=0)
    tri = (lax.broadcasted_iota(jnp.int32, (KF, KF), 0) >=
           lax.broadcasted_iota(jnp.int32, (KF, KF), 1)).astype(jnp.float32)
    F = jax.lax.dot_general(fg, tri, (((1,), (0,)), ((), ())),
                            preferred_element_type=jnp.float32)
    r = KF // KB
    m2 = (r * lax.broadcasted_iota(jnp.int32, (KB, KF), 0) >=
          lax.broadcasted_iota(jnp.int32, (KB, KF), 1)).astype(jnp.float32)
    Nbg = jax.lax.dot_general(bg, m2, (((1,), (0,)), ((), ())),
                              preferred_element_type=jnp.float32)
    N = F + Nbg
    gts = F[:, 0:1]
    denom = gts + N - F
    J = jnp.where(denom > 0, 1.0 - (gts - F) / jnp.maximum(denom, 1.0), 0.0)
    losses = (jnp.sum(J, axis=1) - 0.5 * J[:, 0]) / float(KF)
    maskv = (gts[:, 0] > 0).astype(jnp.float32)
    val = jnp.sum(losses * maskv) / jnp.sum(maskv)
    out_ref[...] = val.reshape(1, 1)


def kernel(probas, labels):
    B, C, H, W = probas.shape
    HW = H * W
    P = B * HW
    NW = 32
    PPW = P // NW
    S = 1024

    pf = probas.reshape(B * C, HW // 128, 128)
    lf = labels.reshape(-1).astype(jnp.int32)

    mesh = plsc.VectorSubcoreMesh(core_axis_name="c", subcore_axis_name="s")
    hist = pl.kernel(
        functools.partial(_sc_histogram_kernel, C, HW, PPW, S),
        mesh=mesh,
        compiler_params=pltpu.CompilerParams(needs_layout_passes=False,
                                             use_tc_tiling_on_sc=True),
        out_type=jax.ShapeDtypeStruct((NW * C * HPC,), jnp.float32),
        scratch_types=[
            pltpu.VMEM((C * S // 128, 128), jnp.float32),
            pltpu.VMEM((C * S // 128, 128), jnp.float32),
            pltpu.VMEM((S,), jnp.int32),
            pltpu.VMEM((S,), jnp.int32),
            pltpu.SemaphoreType.DMA,
            pltpu.SemaphoreType.DMA,
        ] + [pltpu.VMEM((HPC,), jnp.float32) for _ in range(C)],
    )(pf, lf)

    h3 = hist.reshape(NW, C, HPC)
    bg = h3[:, :, :KB * 16].reshape(NW, C, KB, 16)
    fgp = h3[:, :, KB * 16:]
    out = pl.pallas_call(
        functools.partial(_finalize_kernel, C),
        out_shape=jax.ShapeDtypeStruct((1, 1), jnp.float32),
    )(bg, fgp)
    return out.reshape(())


# speedup vs baseline: 1.1045x; 1.1045x over previous
import functools

import jax
import jax.numpy as jnp
from jax import lax
from jax.experimental import pallas as pl
from jax.experimental.pallas import tpu as pltpu
from jax.experimental.pallas import tpu_sc as plsc

KF = 512
KB = 128
HPC = KB * 16 + KF
LANES = 16


def _sc_histogram_kernel(C, HW, PPW, S, pf_hbm, lf_hbm, out_hbm,
                         prob_a, prob_b, lab_a, lab_b, sem_a, sem_b, *hists):
    cid = lax.axis_index("c")
    sid = lax.axis_index("s")
    wid = sid * 2 + cid
    base = wid * PPW
    wpb = HW // PPW
    b = wid // wpb
    off = base - b * HW

    row0 = b * C

    for hc in hists:
        def zero_body(i, _, hc=hc):
            hc[pl.ds(i * LANES, LANES)] = jnp.zeros((LANES,), jnp.float32)
            return 0

        lax.fori_loop(0, HPC // LANES, zero_body, 0)

    def issue(t, prob_v, lab_v, sem):
        pltpu.async_copy(lf_hbm.at[pl.ds(base + t * S, S)], lab_v, sem)
        pltpu.async_copy(
            pf_hbm.at[pl.ds(row0, C), pl.ds(off + t * S, S)], prob_v, sem)

    def drain(prob_v, lab_v, sem):
        pltpu.make_async_copy(lf_hbm.at[pl.ds(0, S)], lab_v, sem).wait()
        pltpu.make_async_copy(
            pf_hbm.at[pl.ds(0, C), pl.ds(0, S)], prob_v, sem).wait()

    ones = jnp.ones((LANES,), jnp.float32)
    lane = lax.broadcasted_iota(jnp.int32, (LANES,), 0)
    nvec = S // LANES

    def compute(prob_v, lab_v):
        def vec_body(v2, _):
            idxs = []
            for u in range(2):
                v = v2 * 2 + u
                lab = lab_v[pl.ds(v * LANES, LANES)]
                for c in range(C):
                    p = prob_v[c, pl.ds(v * LANES, LANES)]
                    fg = lab == c
                    bb = (p * float(KF)).astype(jnp.int32)
                    idx_bg = ((bb >> 2) << 4) + lane
                    idx_fg = (KB * 16 + KF - 1) - bb
                    idxs.append((c, jnp.where(fg, idx_fg, idx_bg)))
            for c, idx in idxs:
                plsc.addupdate_scatter(hists[c], [idx], ones)
            return 0

        lax.fori_loop(0, nvec // 2, vec_body, 0)

    NCH = PPW // S
    issue(0, prob_a, lab_a, sem_a)

    def pair_body(i, _):
        t0 = 2 * i
        issue(t0 + 1, prob_b, lab_b, sem_b)
        drain(prob_a, lab_a, sem_a)
        compute(prob_a, lab_a)

        @pl.when(t0 + 2 < NCH)
        def _prefetch():
            issue(t0 + 2, prob_a, lab_a, sem_a)

        drain(prob_b, lab_b, sem_b)
        compute(prob_b, lab_b)
        return 0

    lax.fori_loop(0, NCH // 2, pair_body, 0)
    for c, hc in enumerate(hists):
        pltpu.sync_copy(hc, out_hbm.at[pl.ds(wid * C * HPC + c * HPC, HPC)])


def _finalize_kernel(C, bg_ref, fg_ref, out_ref):
    bgs = jnp.sum(bg_ref[...], axis=0)
    bg = jnp.sum(bgs, axis=-1)
    fg = jnp.sum(fg_ref[...], axis=0)
    tri = (lax.broadcasted_iota(jnp.int32, (KF, KF), 0) >=
           lax.broadcasted_iota(jnp.int32, (KF, KF), 1)).astype(jnp.float32)
    F = jax.lax.dot_general(fg, tri, (((1,), (0,)), ((), ())),
                            preferred_element_type=jnp.float32)
    r = KF // KB
    m2 = (r * lax.broadcasted_iota(jnp.int32, (KB, KF), 0) >=
          lax.broadcasted_iota(jnp.int32, (KB, KF), 1)).astype(jnp.float32)
    Nbg = jax.lax.dot_general(bg, m2, (((1,), (0,)), ((), ())),
                              preferred_element_type=jnp.float32)
    N = F + Nbg
    gts = F[:, 0:1]
    denom = gts + N - F
    J = jnp.where(denom > 0, 1.0 - (gts - F) / jnp.maximum(denom, 1.0), 0.0)
    losses = (jnp.sum(J, axis=1) - 0.5 * J[:, 0]) / float(KF)
    maskv = (gts[:, 0] > 0).astype(jnp.float32)
    val = jnp.sum(losses * maskv) / jnp.sum(maskv)
    out_ref[...] = val.reshape(1, 1)


def kernel(probas, labels):
    B, C, H, W = probas.shape
    HW = H * W
    P = B * HW
    NW = 32
    PPW = P // NW
    S = 1024

    pf = probas.reshape(B * C, HW)
    lf = labels.reshape(-1).astype(jnp.int32)

    mesh = plsc.VectorSubcoreMesh(core_axis_name="c", subcore_axis_name="s")
    hist = pl.kernel(
        functools.partial(_sc_histogram_kernel, C, HW, PPW, S),
        mesh=mesh,
        compiler_params=pltpu.CompilerParams(needs_layout_passes=False,
                                             use_tc_tiling_on_sc=False),
        out_type=jax.ShapeDtypeStruct((NW * C * HPC,), jnp.float32),
        scratch_types=[
            pltpu.VMEM((C, S), jnp.float32),
            pltpu.VMEM((C, S), jnp.float32),
            pltpu.VMEM((S,), jnp.int32),
            pltpu.VMEM((S,), jnp.int32),
            pltpu.SemaphoreType.DMA,
            pltpu.SemaphoreType.DMA,
        ] + [pltpu.VMEM((HPC,), jnp.float32) for _ in range(C)],
    )(pf, lf)

    h3 = hist.reshape(NW, C, HPC)
    bg = h3[:, :, :KB * 16].reshape(NW, C, KB, 16)
    fgp = h3[:, :, KB * 16:]
    out = pl.pallas_call(
        functools.partial(_finalize_kernel, C),
        out_shape=jax.ShapeDtypeStruct((1, 1), jnp.float32),
    )(bg, fgp)
    return out.reshape(())
